# Initial kernel scaffold; baseline (speedup 1.0000x reference)
import jax
import jax.numpy as jnp
from jax.experimental import pallas as pl


def kernel(batch, edge_index, token_idx, x, edge_weight, heur, W_proj, b_proj, W1_cn, b1_cn, W2_cn, b2_cn, W1_aa, b1_aa, W2_aa, b2_aa, W1_ppr, b1_ppr, W2_ppr, b2_ppr, W1_drnl, b1_drnl, W2_drnl, b2_drnl, Ws1, bs1, Ws2, bs2, cls_tok, Wq, Wk, Wv, Wo):
    raise NotImplementedError("write your pallas kernel here")


# speedup vs baseline: 2.3851x; 2.3851x over previous
import functools

import numpy as np
import jax
import jax.numpy as jnp
from jax import lax
from jax.experimental import pallas as pl
from jax.experimental.pallas import tpu as pltpu
from jax.experimental.pallas import tpu_sc as plsc

NS = 16
CH = 128


def _proj_block(x_ref, w_ref, b_ref, o_ref):
    y = jnp.dot(x_ref[...], w_ref[...], preferred_element_type=jnp.float32)
    y = y + b_ref[...]
    hd = y.shape[1] // 2
    o_ref[0] = y[:, :hd]
    o_ref[1] = y[:, hd:]


def _proj(x, W, b):
    n, f = x.shape
    dim = W.shape[1]
    hd = dim // 2
    rb = 1000
    grid = n // rb
    out = pl.pallas_call(
        _proj_block,
        grid=(grid,),
        in_specs=[
            pl.BlockSpec((rb, f), lambda i: (i, 0)),
            pl.BlockSpec((f, dim), lambda i: (0, 0)),
            pl.BlockSpec((dim,), lambda i: (0,)),
        ],
        out_specs=pl.BlockSpec((2, rb, hd), lambda i: (0, i, 0)),
        out_shape=jax.ShapeDtypeStruct((2, n, hd), jnp.float32),
    )(x, W, b)
    return out.reshape(2 * n, hd)


def _make_round(n, hd, nchunk, with_mix):
    rows_t = (n // (NS * 8)) * 8
    tail_base = NS * rows_t
    tail_cnt = n - tail_base

    mesh = plsc.VectorSubcoreMesh(core_axis_name="c", subcore_axis_name="s")

    @functools.partial(
        pl.kernel,
        mesh=mesh,
        out_type=jax.ShapeDtypeStruct((2 * n, hd), jnp.float32),
        compiler_params=pltpu.CompilerParams(use_tc_tiling_on_sc=False),
        scratch_types=[
            pltpu.VMEM((nchunk, CH), jnp.int32),
            pltpu.VMEM((nchunk, CH), jnp.int32),
            pltpu.VMEM((nchunk, CH), jnp.float32),
            pltpu.VMEM((CH, hd), jnp.float32),
            pltpu.VMEM((CH, hd), jnp.float32),
            pltpu.VMEM((CH, hd), jnp.float32),
            pltpu.VMEM_SHARED((n, hd), jnp.float32),
            pltpu.SemaphoreType.DMA,
        ],
    )
    def round_kernel(h_in, srcb, dstb, wb, mix, out,
                     srcv, dstv, wv, rows, va, vb, acc, sem):
        c = lax.axis_index("c")
        s = lax.axis_index("s")
        w_id = c * NS + s

        pltpu.sync_copy(srcb.at[w_id], srcv)
        pltpu.sync_copy(dstb.at[s], dstv)
        pltpu.sync_copy(wb.at[s], wv)

        def _zrow(j, carry):
            for q in range(hd // 16):
                rows[j, pl.ds(q * 16, 16)] = jnp.zeros((16,), jnp.float32)
            return carry
        lax.fori_loop(0, CH, _zrow, 0)

        def _zero_span(base, cnt):
            off = 0
            while off < cnt:
                c2 = min(CH, cnt - off)
                pltpu.sync_copy(rows.at[pl.ds(0, c2)],
                                acc.at[pl.ds(base + off, c2)])
                off += c2

        _zero_span(s * rows_t, rows_t)

        @pl.when(s == 0)
        def _zero_tail():
            _zero_span(tail_base, tail_cnt)
        plsc.subcore_barrier()

        def chunk_body(j, carry):
            pltpu.async_copy(h_in.at[srcv.at[j]], rows, sem).wait()

            def group_body(g, c2):
                w16 = wv[j, pl.ds(g * 16, 16)]
                for l in range(16):
                    ee = g * 16 + l
                    wvec = jnp.full((16,), w16[l], jnp.float32)
                    for q in range(hd // 16):
                        sl = pl.ds(q * 16, 16)
                        rows[ee, sl] = rows[ee, sl] * wvec
                return c2
            lax.fori_loop(0, CH // 16, group_body, 0)
            pltpu.sync_copy(rows, acc.at[dstv.at[j]], add=True)
            return carry
        lax.fori_loop(0, nchunk, chunk_body, 0)
        plsc.subcore_barrier()

        def _write_span(span_base, span_cnt):
            off = 0
            while off < span_cnt:
                cnt = min(CH, span_cnt - off)
                base = span_base + off
                pltpu.sync_copy(acc.at[pl.ds(base, cnt)], va.at[pl.ds(0, cnt)])
                if with_mix:
                    pltpu.sync_copy(mix.at[pl.ds(c * n + base, cnt)],
                                    vb.at[pl.ds(0, cnt)])

                    def mix_body(i, carry):
                        for q in range(hd // 16):
                            sl = pl.ds(q * 16, 16)
                            va[i, sl] = (va[i, sl] + vb[i, sl]) * 0.5
                        return carry
                    lax.fori_loop(0, cnt, mix_body, 0)
                pltpu.sync_copy(va.at[pl.ds(0, cnt)],
                                out.at[pl.ds(c * n + base, cnt)])
                off += cnt

        _write_span(s * rows_t, rows_t)

        @pl.when(s == 0)
        def _write_tail():
            _write_span(tail_base, tail_cnt)

    return round_kernel


def _make_gather(n, hd, nchunk, tok_total):
    per_tile = nchunk * CH
    mesh = plsc.VectorSubcoreMesh(core_axis_name="c", subcore_axis_name="s")

    @functools.partial(
        pl.kernel,
        mesh=mesh,
        out_type=jax.ShapeDtypeStruct((2 * tok_total, hd), jnp.float32),
        compiler_params=pltpu.CompilerParams(use_tc_tiling_on_sc=False),
        scratch_types=[
            pltpu.VMEM((nchunk, CH), jnp.int32),
            pltpu.VMEM((CH, hd), jnp.float32),
            pltpu.SemaphoreType.DMA,
        ],
    )
    def gather_kernel(tab, idxb, out, idxv, rows, sem):
        c = lax.axis_index("c")
        s = lax.axis_index("s")
        w_id = c * NS + s
        pltpu.sync_copy(idxb.at[w_id], idxv)

        def body(j, carry):
            pltpu.async_copy(tab.at[idxv.at[j]], rows, sem).wait()
            pltpu.sync_copy(
                rows, out.at[pl.ds(c * tok_total + s * per_tile + j * CH, CH)])
            return carry
        lax.fori_loop(0, nchunk, body, 0)

    return gather_kernel


def _make_dense(pb, tpp, dim, d, nheads):
    rb = pb * tpp
    dh = d // nheads
    scale = 1.0 / np.sqrt(dh)
    hd = dim // 2

    def body(t0_ref, t1_ref, hb_ref, w1a_ref, b1a_ref,
             w2cn_ref, b2cn_ref, w2aa_ref, b2aa_ref,
             w2pp_ref, b2pp_ref, w2dr_ref, b2dr_ref,
             ws1_ref, bs1_ref, ws2_ref, bs2_ref,
             cls_ref, wq_ref, wk_ref, wv_ref, wo_ref, o_ref):
        f32 = jnp.float32
        dot = lambda a, b: jnp.dot(a, b, preferred_element_type=f32)
        hb = hb_ref[...]
        u = jnp.maximum(dot(hb, w1a_ref[...]) + b1a_ref[...], 0.0)
        parts = []
        for j, (w2, b2) in enumerate(
                [(w2cn_ref, b2cn_ref), (w2aa_ref, b2aa_ref),
                 (w2pp_ref, b2pp_ref), (w2dr_ref, b2dr_ref)]):
            parts.append(dot(u[:, j * dim:(j + 1) * dim], w2[...]) + b2[...])
        sconc = jnp.concatenate(parts, axis=-1)
        struct = dot(jnp.maximum(dot(sconc, ws1_ref[...]) + bs1_ref[...], 0.0),
                     ws2_ref[...]) + bs2_ref[...]
        htok = jnp.concatenate([t0_ref[...], t1_ref[...], struct], axis=-1)
        kmat = dot(htok, wk_ref[...])
        vmat = dot(htok, wv_ref[...])
        cls2 = cls_ref[...][None, :]
        kcls = dot(cls2, wk_ref[...])
        vcls = dot(cls2, wv_ref[...])

        qin_rows = []
        for p in range(pb):
            qin_rows.append(cls2)
            qin_rows.append(htok[p * tpp:p * tpp + 1])
            qin_rows.append(htok[p * tpp + 1:p * tpp + 2])
        qin = jnp.concatenate(qin_rows, axis=0)
        q = dot(qin, wq_ref[...])

        col = lax.broadcasted_iota(jnp.int32, (nheads, d), 1)
        row = lax.broadcasted_iota(jnp.int32, (nheads, d), 0)
        msk = (col // dh == row).astype(f32)
        att_rows = []
        for p in range(pb):
            kp = jnp.concatenate([kcls, kmat[p * tpp:(p + 1) * tpp]], axis=0)
            vp = jnp.concatenate([vcls, vmat[p * tpp:(p + 1) * tpp]], axis=0)
            qp = q[3 * p:3 * p + 3]
            qb = jnp.concatenate(
                [qp * msk[h][None, :] for h in range(nheads)], axis=0)
            sc = lax.dot_general(qb, kp, (((1,), (1,)), ((), ())),
                                 preferred_element_type=f32) * scale
            sc = sc - jnp.max(sc, axis=-1, keepdims=True)
            es = jnp.exp(sc)
            at = es / jnp.sum(es, axis=-1, keepdims=True)
            pmat = dot(at, vp)
            ob = jnp.zeros((3, d), f32)
            for h in range(nheads):
                ob = ob + pmat[3 * h:3 * h + 3] * msk[h][None, :]
            att_rows.append(ob)
        att = jnp.concatenate(att_rows, axis=0)
        outr = dot(att, wo_ref[...]) + qin
        fin = []
        for p in range(pb):
            fin.append(jnp.concatenate(
                [outr[3 * p + 1:3 * p + 2], outr[3 * p + 2:3 * p + 3],
                 outr[3 * p:3 * p + 1]], axis=-1))
        o_ref[...] = jnp.concatenate(fin, axis=0)

    def full(i):
        return pl.BlockSpec(None, None)

    def call(bsz, tok0, tok1, heurf, w1a, b1a, w2s, ws1, bs1, ws2, bs2,
             cls_tok, wq, wk, wv, wo):
        grid = bsz // pb
        blk = lambda shape: pl.BlockSpec(shape, lambda i: tuple(0 for _ in shape))
        in_specs = [
            pl.BlockSpec((rb, hd), lambda i: (i, 0)),
            pl.BlockSpec((rb, hd), lambda i: (i, 0)),
            pl.BlockSpec((rb, 8), lambda i: (i, 0)),
            blk(w1a.shape), blk(b1a.shape),
        ]
        args = [tok0, tok1, heurf, w1a, b1a]
        for w2, b2 in w2s:
            in_specs += [blk(w2.shape), blk(b2.shape)]
            args += [w2, b2]
        for a in (ws1, bs1, ws2, bs2, cls_tok, wq, wk, wv, wo):
            in_specs.append(blk(a.shape))
            args.append(a)
        return pl.pallas_call(
            body,
            grid=(grid,),
            in_specs=in_specs,
            out_specs=pl.BlockSpec((pb, 3 * d), lambda i: (i, 0)),
            out_shape=jax.ShapeDtypeStruct((bsz, 3 * d), jnp.float32),
        )(*args)

    return call


def kernel(batch, edge_index, token_idx, x, edge_weight, heur,
           W_proj, b_proj,
           W1_cn, b1_cn, W2_cn, b2_cn, W1_aa, b1_aa, W2_aa, b2_aa,
           W1_ppr, b1_ppr, W2_ppr, b2_ppr, W1_drnl, b1_drnl, W2_drnl, b2_drnl,
           Ws1, bs1, Ws2, bs2, cls_tok, Wq, Wk, Wv, Wo):
    n, f = x.shape
    e = edge_index.shape[1]
    bsz, m = token_idx.shape
    dim = W_proj.shape[1]
    hd = dim // 2
    d = 2 * dim
    nheads = 4
    tpp = m + 2

    et = ((e + NS * CH - 1) // (NS * CH)) * CH
    nchunk = et // CH
    e_pad = NS * et
    src = edge_index[0]
    dst = edge_index[1]
    pad = e_pad - e
    srcp = jnp.pad(src, (0, pad))
    dstp = jnp.pad(dst, (0, pad))
    wp = jnp.pad(edge_weight, (0, pad))
    srcb = jnp.concatenate([srcp, srcp + n]).reshape(2 * NS, nchunk, CH)
    dstb = dstp.reshape(NS, nchunk, CH)
    wb = wp.reshape(NS, nchunk, CH)

    feats2 = _proj(x, W_proj, b_proj)

    round_plain = _make_round(n, hd, nchunk, with_mix=False)
    round_mix = _make_round(n, hd, nchunk, with_mix=True)
    h = round_plain(feats2, srcb, dstb, wb, feats2)
    h = round_plain(h, srcb, dstb, wb, feats2)
    xn = round_mix(h, srcb, dstb, wb, feats2)

    idx_all = jnp.concatenate(
        [batch[0][:, None], batch[1][:, None], token_idx], axis=1).reshape(-1)
    tok_total = bsz * tpp
    g_nchunk = tok_total // (NS * CH)
    idxb = jnp.concatenate([idx_all, idx_all + n]).reshape(
        2 * NS, g_nchunk, CH)
    gather_k = _make_gather(n, hd, g_nchunk, tok_total)
    tok = gather_k(xn, idxb)
    tok0 = tok[:tok_total]
    tok1 = tok[tok_total:]

    heurf = heur.reshape(tok_total, 8)
    w1a = jnp.zeros((8, 4 * dim), jnp.float32)
    for j, w1 in enumerate((W1_cn, W1_aa, W1_ppr, W1_drnl)):
        w1a = w1a.at[2 * j:2 * j + 2, j * dim:(j + 1) * dim].set(w1)
    b1a = jnp.concatenate([b1_cn, b1_aa, b1_ppr, b1_drnl])
    w2s = [(W2_cn, b2_cn), (W2_aa, b2_aa), (W2_ppr, b2_ppr), (W2_drnl, b2_drnl)]
    dense = _make_dense(8, tpp, dim, d, nheads)
    return dense(bsz, tok0, tok1, heurf, w1a, b1a, w2s,
                 Ws1, bs1, Ws2, bs2, cls_tok, Wq, Wk, Wv, Wo)


# speedup vs baseline: 4.2770x; 1.7932x over previous
import functools

import numpy as np
import jax
import jax.numpy as jnp
from jax import lax
from jax.experimental import pallas as pl
from jax.experimental.pallas import tpu as pltpu
from jax.experimental.pallas import tpu_sc as plsc

NS = 16
CH = 128


def _proj_block(x_ref, w_ref, b_ref, o_ref):
    y = jnp.dot(x_ref[...], w_ref[...], preferred_element_type=jnp.float32)
    y = y + b_ref[...]
    hd = y.shape[1] // 2
    o_ref[0] = y[:, :hd]
    o_ref[1] = y[:, hd:]


def _proj(x, W, b):
    n, f = x.shape
    dim = W.shape[1]
    hd = dim // 2
    rb = 1000
    grid = n // rb
    out = pl.pallas_call(
        _proj_block,
        grid=(grid,),
        in_specs=[
            pl.BlockSpec((rb, f), lambda i: (i, 0)),
            pl.BlockSpec((f, dim), lambda i: (0, 0)),
            pl.BlockSpec((dim,), lambda i: (0,)),
        ],
        out_specs=pl.BlockSpec((2, rb, hd), lambda i: (0, i, 0)),
        out_shape=jax.ShapeDtypeStruct((2, n, hd), jnp.float32),
    )(x, W, b)
    return out.reshape(2 * n, hd)


def _make_round(n, hd, nchunk, with_mix):
    rows_t = (n // (NS * 8)) * 8
    tail_base = NS * rows_t
    tail_cnt = n - tail_base

    mesh = plsc.VectorSubcoreMesh(core_axis_name="c", subcore_axis_name="s")

    @functools.partial(
        pl.kernel,
        mesh=mesh,
        out_type=jax.ShapeDtypeStruct((2 * n, hd), jnp.float32),
        compiler_params=pltpu.CompilerParams(use_tc_tiling_on_sc=False),
        scratch_types=[
            pltpu.VMEM((nchunk, CH), jnp.int32),
        ] + [pltpu.VMEM((CH, hd), jnp.float32)] * 8
          + [pltpu.VMEM((CH,), jnp.int32)] * 8
          + [pltpu.VMEM((CH,), jnp.float32)] * 8 + [
            pltpu.VMEM_SHARED((n, hd), jnp.float32),
        ] + [pltpu.SemaphoreType.DMA] * 16,
    )
    def round_kernel(h_in, srcb, dstb, wb, mix, out,
                     dstv, g0, g1, g2, g3, s0, s1, s2, s3,
                     i0, i1, i2, i3, i4, i5, i6, i7,
                     w0, w1, w2, w3, w4, w5, w6, w7,
                     acc, gm0, gm1, gm2, gm3, sm0, sm1, sm2, sm3,
                     q0, q1, q2, q3, q4, q5, q6, q7):
        gbufs = [g0, g1, g2, g3]
        sbufs = [s0, s1, s2, s3]
        srcs = [i0, i1, i2, i3, i4, i5, i6, i7]
        wslots = [w0, w1, w2, w3, w4, w5, w6, w7]
        gsems = [gm0, gm1, gm2, gm3]
        ssems = [sm0, sm1, sm2, sm3]
        isems = [q0, q1, q2, q3, q4, q5, q6, q7]
        rows, rows2, va = g0, s0, g1
        c = lax.axis_index("c")
        s = lax.axis_index("s")
        w_id = c * NS + s

        pltpu.sync_copy(dstb.at[s], dstv)

        def _zrow(j, carry):
            for q in range(hd // 16):
                rows[j, pl.ds(q * 16, 16)] = jnp.zeros((16,), jnp.float32)
            return carry
        lax.fori_loop(0, CH, _zrow, 0)

        def _zero_span(base, cnt):
            off = 0
            while off < cnt:
                c2 = min(CH, cnt - off)
                pltpu.sync_copy(rows.at[pl.ds(0, c2)],
                                acc.at[pl.ds(base + off, c2)])
                off += c2

        _zero_span(s * rows_t, rows_t)

        @pl.when(s == 0)
        def _zero_tail():
            _zero_span(tail_base, tail_cnt)
        plsc.subcore_barrier()

        def _scale(buf_in, buf_out, wq):
            def _group(g, carry):
                w16 = wq[pl.ds(g * 16, 16)]
                for l in range(16):
                    ee = g * 16 + l
                    wvec = jnp.full((16,), w16[l], jnp.float32)
                    for q in range(hd // 16):
                        sl = pl.ds(q * 16, 16)
                        buf_out[ee, sl] = buf_in[ee, sl] * wvec
                return carry
            lax.fori_loop(0, CH // 16, _group, 0)

        def _idx_issue(slot, j):
            pltpu.async_copy(srcb.at[w_id, j], srcs[slot], isems[slot])
            pltpu.async_copy(wb.at[s, j], wslots[slot], isems[slot])

        def _idx_wait(slot, j):
            pltpu.make_async_copy(
                srcb.at[w_id, j], srcs[slot], isems[slot]).wait()
            pltpu.make_async_copy(
                wb.at[s, j], wslots[slot], isems[slot]).wait()

        def _drain_scatter(b):
            pltpu.make_async_copy(
                sbufs[b], acc.at[dstv.at[0]], ssems[b]).wait()

        for q in range(8):
            _idx_issue(q, q)
        for b in range(4):
            _idx_wait(b, b)
            pltpu.async_copy(h_in.at[srcs[b]], gbufs[b], gsems[b])

        def oct_body(i2, carry):
            for k in range(8):
                j = 8 * i2 + k
                b = k % 4
                pltpu.make_async_copy(
                    h_in.at[srcs[k]], gbufs[b], gsems[b]).wait()
                if k >= 4:
                    _drain_scatter(b)
                else:
                    @pl.when(i2 > 0)
                    def _d(b=b):
                        _drain_scatter(b)
                _scale(gbufs[b], sbufs[b], wslots[k])
                pltpu.async_copy(sbufs[b], acc.at[dstv.at[j]], ssems[b],
                                 add=True)

                @pl.when(j + 8 < nchunk)
                def _refill(k=k, j=j):
                    _idx_issue(k, j + 8)

                @pl.when(j + 4 < nchunk)
                def _gnext(b=b, k=k, j=j):
                    k4 = (k + 4) % 8
                    _idx_wait(k4, j + 4)
                    pltpu.async_copy(h_in.at[srcs[k4]], gbufs[b], gsems[b])
            return carry
        lax.fori_loop(0, nchunk // 8, oct_body, 0)
        for b in range(4):
            _drain_scatter(b)
        plsc.subcore_barrier()

        def _write_span(span_base, span_cnt):
            off = 0
            while off < span_cnt:
                cnt = min(CH, span_cnt - off)
                base = span_base + off
                pltpu.sync_copy(acc.at[pl.ds(base, cnt)], va.at[pl.ds(0, cnt)])
                if with_mix:
                    pltpu.sync_copy(mix.at[pl.ds(c * n + base, cnt)],
                                    rows.at[pl.ds(0, cnt)])

                    def _mix_row(i, carry):
                        for q in range(hd // 16):
                            sl = pl.ds(q * 16, 16)
                            rows2[i, sl] = (va[i, sl] + rows[i, sl]) * 0.5
                        return carry
                    lax.fori_loop(0, cnt, _mix_row, 0)
                    pltpu.sync_copy(rows2.at[pl.ds(0, cnt)],
                                    out.at[pl.ds(c * n + base, cnt)])
                else:
                    pltpu.sync_copy(va.at[pl.ds(0, cnt)],
                                    out.at[pl.ds(c * n + base, cnt)])
                off += cnt

        _write_span(s * rows_t, rows_t)

        @pl.when(s == 0)
        def _write_tail():
            _write_span(tail_base, tail_cnt)

    return round_kernel


def _make_gather(n, hd, nchunk, tok_total):
    per_tile = nchunk * CH
    mesh = plsc.VectorSubcoreMesh(core_axis_name="c", subcore_axis_name="s")

    @functools.partial(
        pl.kernel,
        mesh=mesh,
        out_type=jax.ShapeDtypeStruct((2 * tok_total, hd), jnp.float32),
        compiler_params=pltpu.CompilerParams(use_tc_tiling_on_sc=False),
        scratch_types=[
            pltpu.VMEM((nchunk, CH), jnp.int32),
            pltpu.VMEM((CH, hd), jnp.float32),
            pltpu.SemaphoreType.DMA,
        ],
    )
    def gather_kernel(tab, idxb, out, idxv, rows, sem):
        c = lax.axis_index("c")
        s = lax.axis_index("s")
        w_id = c * NS + s
        pltpu.sync_copy(idxb.at[w_id], idxv)

        def body(j, carry):
            pltpu.async_copy(tab.at[idxv.at[j]], rows, sem).wait()
            pltpu.sync_copy(
                rows, out.at[pl.ds(c * tok_total + s * per_tile + j * CH, CH)])
            return carry
        lax.fori_loop(0, nchunk, body, 0)

    return gather_kernel


def _make_struct(rb, dim):
    def body(hb_ref, w1a_ref, b1a_ref,
             w2cn_ref, b2cn_ref, w2aa_ref, b2aa_ref,
             w2pp_ref, b2pp_ref, w2dr_ref, b2dr_ref,
             ws1_ref, bs1_ref, ws2_ref, bs2_ref, o_ref):
        f32 = jnp.float32
        dot = lambda a, b: jnp.dot(a, b, preferred_element_type=f32)
        u = jnp.maximum(dot(hb_ref[...], w1a_ref[...]) + b1a_ref[...], 0.0)
        parts = []
        for j, (w2, b2) in enumerate(
                [(w2cn_ref, b2cn_ref), (w2aa_ref, b2aa_ref),
                 (w2pp_ref, b2pp_ref), (w2dr_ref, b2dr_ref)]):
            parts.append(dot(u[:, j * dim:(j + 1) * dim], w2[...]) + b2[...])
        sconc = jnp.concatenate(parts, axis=-1)
        o_ref[...] = dot(
            jnp.maximum(dot(sconc, ws1_ref[...]) + bs1_ref[...], 0.0),
            ws2_ref[...]) + bs2_ref[...]

    def call(tok_total, heurf, w1a, b1a, w2s, ws1, bs1, ws2, bs2):
        grid = tok_total // rb
        blk = lambda a: pl.BlockSpec(a.shape, lambda i: tuple(0 for _ in a.shape))
        in_specs = [pl.BlockSpec((rb, 8), lambda i: (i, 0)),
                    blk(w1a), blk(b1a)]
        args = [heurf, w1a, b1a]
        for w2, b2 in w2s:
            in_specs += [blk(w2), blk(b2)]
            args += [w2, b2]
        for a in (ws1, bs1, ws2, bs2):
            in_specs.append(blk(a))
            args.append(a)
        return pl.pallas_call(
            body,
            grid=(grid,),
            in_specs=in_specs,
            out_specs=pl.BlockSpec((rb, dim), lambda i: (i, 0)),
            out_shape=jax.ShapeDtypeStruct((tok_total, dim), jnp.float32),
        )(*args)

    return call


def _make_attn(pb, tpp, dim, d, nheads, tok_total):
    rb = pb * tpp
    dh = d // nheads
    tk = tpp + 1
    nq = 3 * nheads
    scale = 1.0 / np.sqrt(dh)
    hd = dim // 2
    off = tok_total // rb

    def body(t0_ref, t1_ref, st_ref, cls_ref, wq_ref, wk_ref, wv_ref,
             wo_ref, o_ref):
        f32 = jnp.float32
        dot = lambda a, b: jnp.dot(a, b, preferred_element_type=f32)
        htok = jnp.concatenate([t0_ref[...], t1_ref[...], st_ref[...]],
                               axis=-1)
        kmat = dot(htok, wk_ref[...])
        vmat = dot(htok, wv_ref[...])
        cls2 = cls_ref[...][None, :]
        kcls = dot(cls2, wk_ref[...])
        vcls = dot(cls2, wv_ref[...])

        qin_rows = []
        for p in range(pb):
            qin_rows.append(cls2)
            qin_rows.append(htok[p * tpp:p * tpp + 1])
            qin_rows.append(htok[p * tpp + 1:p * tpp + 2])
        qin = jnp.concatenate(qin_rows, axis=0)
        q = dot(qin, wq_ref[...])

        col = lax.broadcasted_iota(jnp.int32, (nheads, d), 1)
        row = lax.broadcasted_iota(jnp.int32, (nheads, d), 0)
        msk = (col // dh == row).astype(f32)

        krows, vrows, qb_rows = [], [], []
        for p in range(pb):
            krows.append(kcls)
            krows.append(kmat[p * tpp:(p + 1) * tpp])
            vrows.append(vcls)
            vrows.append(vmat[p * tpp:(p + 1) * tpp])
            qp = q[3 * p:3 * p + 3]
            for h in range(nheads):
                qb_rows.append(qp * msk[h][None, :])
        k_all = jnp.concatenate(krows, axis=0)
        v_all = jnp.concatenate(vrows, axis=0)
        qb = jnp.concatenate(qb_rows, axis=0)
        s_all = lax.dot_general(qb, k_all, (((1,), (1,)), ((), ())),
                                preferred_element_type=f32) * scale
        rowp = lax.broadcasted_iota(jnp.int32, (pb * nq, pb * tk), 0) // nq
        colp = lax.broadcasted_iota(jnp.int32, (pb * nq, pb * tk), 1) // tk
        s_all = jnp.where(rowp == colp, s_all, jnp.float32(-1e30))
        s_all = s_all - jnp.max(s_all, axis=-1, keepdims=True)
        es = jnp.exp(s_all)
        at = es / jnp.sum(es, axis=-1, keepdims=True)
        pmat = dot(at, v_all)
        att_rows = []
        for p in range(pb):
            ob = pmat[p * nq:p * nq + 3] * msk[0][None, :]
            for h in range(1, nheads):
                base = p * nq + 3 * h
                ob = ob + pmat[base:base + 3] * msk[h][None, :]
            att_rows.append(ob)
        att = jnp.concatenate(att_rows, axis=0)
        outr = dot(att, wo_ref[...]) + qin
        fin = []
        for p in range(pb):
            fin.append(jnp.concatenate(
                [outr[3 * p + 1:3 * p + 2], outr[3 * p + 2:3 * p + 3],
                 outr[3 * p:3 * p + 1]], axis=-1))
        o_ref[...] = jnp.concatenate(fin, axis=0)

    def call(bsz, tok, struct, cls_tok, wq, wk, wv, wo):
        grid = bsz // pb
        blk = lambda a: pl.BlockSpec(a.shape, lambda i: tuple(0 for _ in a.shape))
        in_specs = [
            pl.BlockSpec((rb, hd), lambda i: (i, 0)),
            pl.BlockSpec((rb, hd), lambda i: (i + off, 0)),
            pl.BlockSpec((rb, dim), lambda i: (i, 0)),
            blk(cls_tok), blk(wq), blk(wk), blk(wv), blk(wo),
        ]
        return pl.pallas_call(
            body,
            grid=(grid,),
            in_specs=in_specs,
            out_specs=pl.BlockSpec((pb, 3 * d), lambda i: (i, 0)),
            out_shape=jax.ShapeDtypeStruct((bsz, 3 * d), jnp.float32),
        )(tok, tok, struct, cls_tok, wq, wk, wv, wo)

    return call


def kernel(batch, edge_index, token_idx, x, edge_weight, heur,
           W_proj, b_proj,
           W1_cn, b1_cn, W2_cn, b2_cn, W1_aa, b1_aa, W2_aa, b2_aa,
           W1_ppr, b1_ppr, W2_ppr, b2_ppr, W1_drnl, b1_drnl, W2_drnl, b2_drnl,
           Ws1, bs1, Ws2, bs2, cls_tok, Wq, Wk, Wv, Wo):
    n, f = x.shape
    e = edge_index.shape[1]
    bsz, m = token_idx.shape
    dim = W_proj.shape[1]
    hd = dim // 2
    d = 2 * dim
    nheads = 4
    tpp = m + 2

    nchunk = -(-e // (NS * CH))
    nchunk = ((nchunk + 7) // 8) * 8
    et = nchunk * CH
    e_pad = NS * et
    src = edge_index[0]
    dst = edge_index[1]
    pad = e_pad - e
    srcp = jnp.pad(src, (0, pad))
    dstp = jnp.pad(dst, (0, pad))
    wp = jnp.pad(edge_weight, (0, pad))
    srcb = jnp.concatenate([srcp, srcp + n]).reshape(2 * NS, nchunk, CH)
    dstb = dstp.reshape(NS, nchunk, CH)
    wb = wp.reshape(NS, nchunk, CH)

    tok_total = bsz * tpp
    heurf = heur.reshape(tok_total, 8)
    w1a = jnp.zeros((8, 4 * dim), jnp.float32)
    for j, w1 in enumerate((W1_cn, W1_aa, W1_ppr, W1_drnl)):
        w1a = w1a.at[2 * j:2 * j + 2, j * dim:(j + 1) * dim].set(w1)
    b1a = jnp.concatenate([b1_cn, b1_aa, b1_ppr, b1_drnl])
    w2s = [(W2_cn, b2_cn), (W2_aa, b2_aa), (W2_ppr, b2_ppr), (W2_drnl, b2_drnl)]
    struct_call = _make_struct(16 * tpp, dim)
    struct = struct_call(tok_total, heurf, w1a, b1a, w2s, Ws1, bs1, Ws2, bs2)

    feats2 = _proj(x, W_proj, b_proj)

    round_plain = _make_round(n, hd, nchunk, with_mix=False)
    round_mix = _make_round(n, hd, nchunk, with_mix=True)
    h = round_plain(feats2, srcb, dstb, wb, feats2)
    h = round_plain(h, srcb, dstb, wb, feats2)
    xn = round_mix(h, srcb, dstb, wb, feats2)

    idx_all = jnp.concatenate(
        [batch[0][:, None], batch[1][:, None], token_idx], axis=1).reshape(-1)
    g_nchunk = tok_total // (NS * CH)
    idxb = jnp.concatenate([idx_all, idx_all + n]).reshape(
        2 * NS, g_nchunk, CH)
    gather_k = _make_gather(n, hd, g_nchunk, tok_total)
    tok = gather_k(xn, idxb)

    attn = _make_attn(16, tpp, dim, d, nheads, tok_total)
    return attn(bsz, tok, struct, cls_tok, Wq, Wk, Wv, Wo)


# speedup vs baseline: 6.7818x; 1.5857x over previous
import functools

import numpy as np
import jax
import jax.numpy as jnp
from jax import lax
from jax.experimental import pallas as pl
from jax.experimental.pallas import tpu as pltpu
from jax.experimental.pallas import tpu_sc as plsc

NS = 16
CH = 128
NB = 2
NSLOT = 6


def _proj_block(x_ref, w_ref, b_ref, o_ref):
    y = jnp.dot(x_ref[...], w_ref[...], preferred_element_type=jnp.float32)
    y = y + b_ref[...]
    hd = y.shape[1] // 2
    o_ref[0] = y[:, :hd]
    o_ref[1] = y[:, hd:]


def _proj(x, W, b):
    n, f = x.shape
    dim = W.shape[1]
    hd = dim // 2
    rb = 1000
    grid = n // rb
    out = pl.pallas_call(
        _proj_block,
        grid=(grid,),
        in_specs=[
            pl.BlockSpec((rb, f), lambda i: (i, 0)),
            pl.BlockSpec((f, dim), lambda i: (0, 0)),
            pl.BlockSpec((dim,), lambda i: (0,)),
        ],
        out_specs=pl.BlockSpec((2, rb, hd), lambda i: (0, i, 0)),
        out_shape=jax.ShapeDtypeStruct((2, n, hd), jnp.float32),
    )(x, W, b)
    return out.reshape(2 * n, hd)


def _make_round(n, hd, nchunk, with_mix):
    rows_t = (n // (NS * 8)) * 8
    tail_base = NS * rows_t
    tail_cnt = n - tail_base
    mesh = plsc.VectorSubcoreMesh(core_axis_name="c", subcore_axis_name="s")

    @functools.partial(
        pl.kernel,
        mesh=mesh,
        out_type=jax.ShapeDtypeStruct((2 * n, hd), jnp.float32),
        compiler_params=pltpu.CompilerParams(use_tc_tiling_on_sc=False),
        scratch_types=(
            [pltpu.VMEM((CH, hd), jnp.float32)] * NB
          + [pltpu.VMEM((CH, hd), jnp.float32)] * 3
          + [pltpu.VMEM((CH,), jnp.int32)] * NSLOT
          + [pltpu.VMEM((CH,), jnp.int32)] * NSLOT
          + [pltpu.VMEM((CH,), jnp.float32)] * NSLOT + [
            pltpu.VMEM_SHARED((n, hd), jnp.float32),
            pltpu.VMEM_SHARED((n, hd), jnp.float32),
        ] + [pltpu.SemaphoreType.DMA] * (NB + 3 + NSLOT)),
    )
    def round_kernel(h_in, srcb, dstb, wb, mix, out,
                     g0, g1, s0, s1, s2,
                     i0, i1, i2, i3, i4, i5,
                     d0, d1, d2, d3, d4, d5,
                     w0, w1, w2, w3, w4, w5,
                     tab, acc,
                     gm0, gm1, sm0, sm1, sm2,
                     q0, q1, q2, q3, q4, q5):
        gbufs = [g0, g1]
        sbufs = [s0, s1, s2]
        srcs = [i0, i1, i2, i3, i4, i5]
        dsts = [d0, d1, d2, d3, d4, d5]
        wslots = [w0, w1, w2, w3, w4, w5]
        gsems = [gm0, gm1]
        ssems = [sm0, sm1, sm2]
        isems = [q0, q1, q2, q3, q4, q5]
        va, vb = g0, s1
        c = lax.axis_index("c")
        s = lax.axis_index("s")
        w_id = c * NS + s
        nh16 = hd // 16
        nh32 = hd // 32

        def _zrow(j, carry):
            for q in range(nh16):
                va[j, pl.ds(q * 16, 16)] = jnp.zeros((16,), jnp.float32)
            return carry
        lax.fori_loop(0, CH, _zrow, 0)

        def _span(fn, base, cnt):
            off = 0
            while off < cnt:
                c2 = min(CH, cnt - off)
                fn(base + off, c2)
                off += c2

        def _tiled(fn):
            _span(fn, s * rows_t, rows_t)

            @pl.when(s == 0)
            def _tail():
                _span(fn, tail_base, tail_cnt)

        def _load(base, c2):
            pltpu.sync_copy(h_in.at[pl.ds(c * n + base, c2)],
                            vb.at[pl.ds(0, c2)])
            pltpu.sync_copy(vb.at[pl.ds(0, c2)], tab.at[pl.ds(base, c2)])

        _tiled(_load)

        def _zero(base, c2):
            pltpu.sync_copy(va.at[pl.ds(0, c2)], acc.at[pl.ds(base, c2)])
        _tiled(_zero)
        plsc.subcore_barrier()

        def _scale(buf_in, buf_out, wq):
            def _group(g, carry):
                w16 = wq[pl.ds(g * 16, 16)]
                for l in range(16):
                    ee = g * 16 + l
                    wvec = jnp.full((16,), w16[l], jnp.float32)
                    for q in range(nh16):
                        sl = pl.ds(q * 16, 16)
                        buf_out[ee, sl] = buf_in[ee, sl] * wvec
                return carry
            lax.fori_loop(0, CH // 16, _group, 0)

        def _idx_issue(slot, j):
            pltpu.async_copy(srcb.at[s, j], srcs[slot], isems[slot])
            pltpu.async_copy(dstb.at[s, j], dsts[slot], isems[slot])
            pltpu.async_copy(wb.at[s, j], wslots[slot], isems[slot])

        def _idx_wait(slot, j):
            pltpu.make_async_copy(
                srcb.at[s, j], srcs[slot], isems[slot]).wait()
            pltpu.make_async_copy(
                dstb.at[s, j], dsts[slot], isems[slot]).wait()
            pltpu.make_async_copy(
                wb.at[s, j], wslots[slot], isems[slot]).wait()

        def _drain_scatter(b):
            pltpu.make_async_copy(
                sbufs[b], acc.at[dsts[0]], ssems[b]).wait()

        for q in range(NB + 1):
            _idx_issue(q, q)
        for b in range(NB):
            _idx_wait(b, b)
            pltpu.async_copy(tab.at[srcs[b]], gbufs[b], gsems[b])

        def sext_body(i2, carry):
            for k in range(NSLOT):
                j = NSLOT * i2 + k
                bg = k % NB
                bs = k % 3
                pltpu.make_async_copy(
                    tab.at[srcs[k]], gbufs[bg], gsems[bg]).wait()
                if k >= 3:
                    _drain_scatter(bs)
                else:
                    @pl.when(i2 > 0)
                    def _d(bs=bs):
                        _drain_scatter(bs)
                @pl.when(j + 3 < nchunk)
                def _refill(k=k, j=j):
                    _idx_issue((k + 3) % NSLOT, j + 3)
                _scale(gbufs[bg], sbufs[bs], wslots[k])
                pltpu.async_copy(sbufs[bs], acc.at[dsts[k]], ssems[bs],
                                 add=True)

                @pl.when(j + NB < nchunk)
                def _gnext(bg=bg, k=k, j=j):
                    kn = (k + NB) % NSLOT
                    _idx_wait(kn, j + NB)
                    pltpu.async_copy(tab.at[srcs[kn]], gbufs[bg], gsems[bg])
            return carry
        lax.fori_loop(0, nchunk // NSLOT, sext_body, 0)
        for b in range(3):
            _drain_scatter(b)
        plsc.subcore_barrier()

        def _write(base, c2):
            pltpu.sync_copy(acc.at[pl.ds(base, c2)], va.at[pl.ds(0, c2)])
            if with_mix:
                pltpu.sync_copy(mix.at[pl.ds(c * n + base, c2)],
                                vb.at[pl.ds(0, c2)])

                def _mx(i, carry):
                    for q in range(nh16):
                        sl = pl.ds(q * 16, 16)
                        s0[i, sl] = (va[i, sl] + vb[i, sl]) * 0.5
                    return carry
                lax.fori_loop(0, c2, _mx, 0)
                pltpu.sync_copy(s0.at[pl.ds(0, c2)],
                                out.at[pl.ds(c * n + base, c2)])
            else:
                pltpu.sync_copy(va.at[pl.ds(0, c2)],
                                out.at[pl.ds(c * n + base, c2)])
        _tiled(_write)

    return round_kernel


def _make_gather(n, hd, nchunk, tok_total):
    per_tile = nchunk * CH
    mesh = plsc.VectorSubcoreMesh(core_axis_name="c", subcore_axis_name="s")

    @functools.partial(
        pl.kernel,
        mesh=mesh,
        out_type=jax.ShapeDtypeStruct((2 * tok_total, hd), jnp.float32),
        compiler_params=pltpu.CompilerParams(use_tc_tiling_on_sc=False),
        scratch_types=[
            pltpu.VMEM((nchunk, CH), jnp.int32),
            pltpu.VMEM((CH, hd), jnp.float32),
            pltpu.SemaphoreType.DMA,
        ],
    )
    def gather_kernel(tab, idxb, out, idxv, rows, sem):
        c = lax.axis_index("c")
        s = lax.axis_index("s")
        w_id = c * NS + s
        pltpu.sync_copy(idxb.at[w_id], idxv)

        def body(j, carry):
            pltpu.async_copy(tab.at[idxv.at[j]], rows, sem).wait()
            pltpu.sync_copy(
                rows, out.at[pl.ds(c * tok_total + s * per_tile + j * CH, CH)])
            return carry
        lax.fori_loop(0, nchunk, body, 0)

    return gather_kernel


def _make_struct(rb, dim):
    def body(hb_ref, w1a_ref, b1a_ref,
             w2cn_ref, b2cn_ref, w2aa_ref, b2aa_ref,
             w2pp_ref, b2pp_ref, w2dr_ref, b2dr_ref,
             ws1_ref, bs1_ref, ws2_ref, bs2_ref, o_ref):
        f32 = jnp.float32
        dot = lambda a, b: jnp.dot(a, b, preferred_element_type=f32)
        u = jnp.maximum(dot(hb_ref[...], w1a_ref[...]) + b1a_ref[...], 0.0)
        parts = []
        for j, (w2, b2) in enumerate(
                [(w2cn_ref, b2cn_ref), (w2aa_ref, b2aa_ref),
                 (w2pp_ref, b2pp_ref), (w2dr_ref, b2dr_ref)]):
            parts.append(dot(u[:, j * dim:(j + 1) * dim], w2[...]) + b2[...])
        sconc = jnp.concatenate(parts, axis=-1)
        o_ref[...] = dot(
            jnp.maximum(dot(sconc, ws1_ref[...]) + bs1_ref[...], 0.0),
            ws2_ref[...]) + bs2_ref[...]

    def call(tok_total, heurf, w1a, b1a, w2s, ws1, bs1, ws2, bs2):
        grid = tok_total // rb
        blk = lambda a: pl.BlockSpec(a.shape, lambda i: tuple(0 for _ in a.shape))
        in_specs = [pl.BlockSpec((rb, 8), lambda i: (i, 0)),
                    blk(w1a), blk(b1a)]
        args = [heurf, w1a, b1a]
        for w2, b2 in w2s:
            in_specs += [blk(w2), blk(b2)]
            args += [w2, b2]
        for a in (ws1, bs1, ws2, bs2):
            in_specs.append(blk(a))
            args.append(a)
        return pl.pallas_call(
            body,
            grid=(grid,),
            in_specs=in_specs,
            out_specs=pl.BlockSpec((rb, dim), lambda i: (i, 0)),
            out_shape=jax.ShapeDtypeStruct((tok_total, dim), jnp.float32),
        )(*args)

    return call


def _make_attn(pb, tpp, dim, d, nheads, tok_total):
    rb = pb * tpp
    dh = d // nheads
    tk = tpp + 1
    nq = 3 * nheads
    scale = 1.0 / np.sqrt(dh)
    hd = dim // 2
    off = tok_total // rb

    def body(t0_ref, t1_ref, st_ref, cls_ref, wq_ref, wk_ref, wv_ref,
             wo_ref, o_ref):
        f32 = jnp.float32
        dot = lambda a, b: jnp.dot(a, b, preferred_element_type=f32)
        htok = jnp.concatenate([t0_ref[...], t1_ref[...], st_ref[...]],
                               axis=-1)
        kmat = dot(htok, wk_ref[...])
        vmat = dot(htok, wv_ref[...])
        cls2 = cls_ref[...][None, :]
        kcls = dot(cls2, wk_ref[...])
        vcls = dot(cls2, wv_ref[...])

        qin_rows = []
        for p in range(pb):
            qin_rows.append(cls2)
            qin_rows.append(htok[p * tpp:p * tpp + 1])
            qin_rows.append(htok[p * tpp + 1:p * tpp + 2])
        qin = jnp.concatenate(qin_rows, axis=0)
        q = dot(qin, wq_ref[...])

        col = lax.broadcasted_iota(jnp.int32, (nheads, d), 1)
        row = lax.broadcasted_iota(jnp.int32, (nheads, d), 0)
        msk = (col // dh == row).astype(f32)

        krows, vrows, qb_rows = [], [], []
        for p in range(pb):
            krows.append(kcls)
            krows.append(kmat[p * tpp:(p + 1) * tpp])
            vrows.append(vcls)
            vrows.append(vmat[p * tpp:(p + 1) * tpp])
            qp = q[3 * p:3 * p + 3]
            for h in range(nheads):
                qb_rows.append(qp * msk[h][None, :])
        k_all = jnp.concatenate(krows, axis=0)
        v_all = jnp.concatenate(vrows, axis=0)
        qb = jnp.concatenate(qb_rows, axis=0)
        s_all = lax.dot_general(qb, k_all, (((1,), (1,)), ((), ())),
                                preferred_element_type=f32) * scale
        rowp = lax.broadcasted_iota(jnp.int32, (pb * nq, pb * tk), 0) // nq
        colp = lax.broadcasted_iota(jnp.int32, (pb * nq, pb * tk), 1) // tk
        s_all = jnp.where(rowp == colp, s_all, jnp.float32(-1e30))
        s_all = s_all - jnp.max(s_all, axis=-1, keepdims=True)
        es = jnp.exp(s_all)
        at = es / jnp.sum(es, axis=-1, keepdims=True)
        pmat = dot(at, v_all)
        att_rows = []
        for p in range(pb):
            ob = pmat[p * nq:p * nq + 3] * msk[0][None, :]
            for h in range(1, nheads):
                base = p * nq + 3 * h
                ob = ob + pmat[base:base + 3] * msk[h][None, :]
            att_rows.append(ob)
        att = jnp.concatenate(att_rows, axis=0)
        outr = dot(att, wo_ref[...]) + qin
        fin = []
        for p in range(pb):
            fin.append(jnp.concatenate(
                [outr[3 * p + 1:3 * p + 2], outr[3 * p + 2:3 * p + 3],
                 outr[3 * p:3 * p + 1]], axis=-1))
        o_ref[...] = jnp.concatenate(fin, axis=0)

    def call(bsz, tok, struct, cls_tok, wq, wk, wv, wo):
        grid = bsz // pb
        blk = lambda a: pl.BlockSpec(a.shape, lambda i: tuple(0 for _ in a.shape))
        in_specs = [
            pl.BlockSpec((rb, hd), lambda i: (i, 0)),
            pl.BlockSpec((rb, hd), lambda i: (i + off, 0)),
            pl.BlockSpec((rb, dim), lambda i: (i, 0)),
            blk(cls_tok), blk(wq), blk(wk), blk(wv), blk(wo),
        ]
        return pl.pallas_call(
            body,
            grid=(grid,),
            in_specs=in_specs,
            out_specs=pl.BlockSpec((pb, 3 * d), lambda i: (i, 0)),
            out_shape=jax.ShapeDtypeStruct((bsz, 3 * d), jnp.float32),
        )(tok, tok, struct, cls_tok, wq, wk, wv, wo)

    return call


def kernel(batch, edge_index, token_idx, x, edge_weight, heur,
           W_proj, b_proj,
           W1_cn, b1_cn, W2_cn, b2_cn, W1_aa, b1_aa, W2_aa, b2_aa,
           W1_ppr, b1_ppr, W2_ppr, b2_ppr, W1_drnl, b1_drnl, W2_drnl, b2_drnl,
           Ws1, bs1, Ws2, bs2, cls_tok, Wq, Wk, Wv, Wo):
    n, f = x.shape
    e = edge_index.shape[1]
    bsz, m = token_idx.shape
    dim = W_proj.shape[1]
    hd = dim // 2
    d = 2 * dim
    nheads = 4
    tpp = m + 2

    nchunk = -(-e // (NS * CH))
    nchunk = ((nchunk + NSLOT - 1) // NSLOT) * NSLOT
    et = nchunk * CH
    e_pad = NS * et
    src = edge_index[0]
    dst = edge_index[1]
    pad = e_pad - e
    srcp = jnp.pad(src, (0, pad))
    dstp = jnp.pad(dst, (0, pad))
    wp = jnp.pad(edge_weight, (0, pad))
    srcb = srcp.reshape(NS, nchunk, CH)
    dstb = dstp.reshape(NS, nchunk, CH)
    wb = wp.reshape(NS, nchunk, CH)

    tok_total = bsz * tpp
    heurf = heur.reshape(tok_total, 8)
    w1a = jnp.zeros((8, 4 * dim), jnp.float32)
    for j, w1 in enumerate((W1_cn, W1_aa, W1_ppr, W1_drnl)):
        w1a = w1a.at[2 * j:2 * j + 2, j * dim:(j + 1) * dim].set(w1)
    b1a = jnp.concatenate([b1_cn, b1_aa, b1_ppr, b1_drnl])
    w2s = [(W2_cn, b2_cn), (W2_aa, b2_aa), (W2_ppr, b2_ppr), (W2_drnl, b2_drnl)]
    struct_call = _make_struct(16 * tpp, dim)
    struct = struct_call(tok_total, heurf, w1a, b1a, w2s, Ws1, bs1, Ws2, bs2)

    feats2 = _proj(x, W_proj, b_proj)

    round_plain = _make_round(n, hd, nchunk, with_mix=False)
    round_mix = _make_round(n, hd, nchunk, with_mix=True)
    h = round_plain(feats2, srcb, dstb, wb, feats2)
    h = round_plain(h, srcb, dstb, wb, feats2)
    xn = round_mix(h, srcb, dstb, wb, feats2)

    idx_all = jnp.concatenate(
        [batch[0][:, None], batch[1][:, None], token_idx], axis=1).reshape(-1)
    g_nchunk = tok_total // (NS * CH)
    idxb = jnp.concatenate([idx_all, idx_all + n]).reshape(
        2 * NS, g_nchunk, CH)
    gather_k = _make_gather(n, hd, g_nchunk, tok_total)
    tok = gather_k(xn, idxb)

    attn = _make_attn(16, tpp, dim, d, nheads, tok_total)
    return attn(bsz, tok, struct, cls_tok, Wq, Wk, Wv, Wo)
